# baseline (speedup 1.0000x reference)
import functools

import numpy as np

import jax
import jax.numpy as jnp
from jax import lax
from jax.experimental import pallas as pl
from jax.experimental.pallas import tpu as pltpu
from jax.experimental.pallas import tpu_sc as plsc

B = 16384
D = 768
C = 5
NC = 2
NS = 16
NW = NC * NS
BPW = B // NW
CHUNK = 16
NCHUNK = BPW // CHUNK
NG = D // 32
CD = CHUNK * D
SW = CD // 2

_PACK_ORDER = np.arange(D).reshape(NG, 2, 16).transpose(0, 2, 1).reshape(-1)


def _gather_sc(user_ids, item_ids, user_table, item_table):
    mesh = plsc.VectorSubcoreMesh(core_axis_name="c", subcore_axis_name="s")

    @functools.partial(
        pl.kernel,
        mesh=mesh,
        out_type=[
            jax.ShapeDtypeStruct((B * D // 2,), jnp.float32),
            jax.ShapeDtypeStruct((B * D // 2,), jnp.float32),
        ],
        scratch_types=[
            pltpu.VMEM((BPW,), jnp.int32),
            pltpu.VMEM((BPW,), jnp.int32),
            pltpu.VMEM((2, CHUNK, D), jnp.float32),
            pltpu.VMEM((2, CHUNK, D), jnp.float32),
            pltpu.VMEM((2 * SW,), jnp.float32),
            pltpu.VMEM((2 * SW,), jnp.float32),
            [pltpu.SemaphoreType.DMA] * 2,
            [pltpu.SemaphoreType.DMA] * 2,
            [pltpu.SemaphoreType.DMA] * 2,
            [pltpu.SemaphoreType.DMA] * 2,
        ],
    )
    def k(uid_hbm, iid_hbm, utab_hbm, itab_hbm, uout_hbm, iout_hbm,
          uidx, iidx, ubuf, ibuf, ustage, istage, gsem_u, gsem_i,
          osem_u, osem_i):
        wid = lax.axis_index("s") * NC + lax.axis_index("c")
        base = wid * BPW
        pltpu.sync_copy(uid_hbm.at[pl.ds(base, BPW)], uidx)
        pltpu.sync_copy(iid_hbm.at[pl.ds(base, BPW)], iidx)

        def start_gather(c, bsl):
            off = pl.multiple_of(c * CHUNK, CHUNK)
            pltpu.async_copy(utab_hbm.at[uidx.at[pl.ds(off, CHUNK)]],
                             ubuf.at[bsl], gsem_u[bsl])
            pltpu.async_copy(itab_hbm.at[iidx.at[pl.ds(off, CHUNK)]],
                             ibuf.at[bsl], gsem_i[bsl])

        def wait_gather(bsl):
            pltpu.make_async_copy(utab_hbm.at[uidx.at[pl.ds(0, CHUNK)]],
                                  ubuf.at[bsl], gsem_u[bsl]).wait()
            pltpu.make_async_copy(itab_hbm.at[iidx.at[pl.ds(0, CHUNK)]],
                                  ibuf.at[bsl], gsem_i[bsl]).wait()

        def start_out(c, bsl):
            off = pl.multiple_of((base + c * CHUNK) * D // 2, SW)
            pltpu.async_copy(ustage.at[pl.ds(bsl * SW, SW)],
                             uout_hbm.at[pl.ds(off, SW)], osem_u[bsl])
            pltpu.async_copy(istage.at[pl.ds(bsl * SW, SW)],
                             iout_hbm.at[pl.ds(off, SW)], osem_i[bsl])

        def wait_out(bsl):
            pltpu.make_async_copy(ustage.at[pl.ds(bsl * SW, SW)],
                                  uout_hbm.at[pl.ds(0, SW)],
                                  osem_u[bsl]).wait()
            pltpu.make_async_copy(istage.at[pl.ds(bsl * SW, SW)],
                                  iout_hbm.at[pl.ds(0, SW)],
                                  osem_i[bsl]).wait()

        def convert_chunk(bsl):
            def row_body(r, _):
                so = bsl * SW + r * (D // 2)

                def cvt(buf, stage):
                    for g in range(NG):
                        lo = buf[bsl, r, pl.ds(32 * g, 16)]
                        hi = buf[bsl, r, pl.ds(32 * g + 16, 16)]
                        blo = lax.bitcast_convert_type(lo, jnp.uint32)
                        bhi = lax.bitcast_convert_type(hi, jnp.uint32)
                        rlo = blo + jnp.uint32(0x8000)
                        rhi = bhi + jnp.uint32(0x8000)
                        word = ((rhi & jnp.uint32(0xFFFF0000))
                                | lax.shift_right_logical(
                                    rlo, jnp.uint32(16)))
                        stage[pl.ds(so + 16 * g, 16)] = (
                            lax.bitcast_convert_type(word, jnp.float32))

                cvt(ubuf, ustage)
                cvt(ibuf, istage)
                return 0

            lax.fori_loop(0, CHUNK, row_body, 0)

        start_gather(0, 0)
        start_gather(1, 1)

        def gbody(g, _):
            for par in range(2):
                c = g * 2 + par
                wait_gather(par)

                @pl.when(c >= 2)
                def _():
                    wait_out(par)

                convert_chunk(par)

                @pl.when(c + 2 < NCHUNK)
                def _():
                    start_gather(c + 2, par)

                start_out(c, par)
            return 0

        lax.fori_loop(0, NCHUNK // 2, gbody, 0)
        wait_out(0)
        wait_out(1)

    return k(user_ids, item_ids, user_table, item_table)


BM = 2048


def _mm_body(u_ref, i_ref, wu_ref, wi_ref, b_ref, o_ref):
    acc = jnp.dot(u_ref[...], wu_ref[...], preferred_element_type=jnp.float32)
    acc = acc + jnp.dot(i_ref[...], wi_ref[...],
                        preferred_element_type=jnp.float32)
    o_ref[...] = acc + b_ref[...]


def _mm_tc(u_rep, i_rep, wu, wi, b2d):
    return pl.pallas_call(
        _mm_body,
        grid=(B // BM,),
        in_specs=[
            pl.BlockSpec((BM, D), lambda m: (m, 0)),
            pl.BlockSpec((BM, D), lambda m: (m, 0)),
            pl.BlockSpec((D, C), lambda m: (0, 0)),
            pl.BlockSpec((D, C), lambda m: (0, 0)),
            pl.BlockSpec((1, C), lambda m: (0, 0)),
        ],
        out_specs=pl.BlockSpec((BM, C), lambda m: (m, 0)),
        out_shape=jax.ShapeDtypeStruct((B, C), jnp.float32),
    )(u_rep, i_rep, wu, wi, b2d)


def kernel(user_ids, item_ids, user_table, item_table, W, b):
    uids = user_ids.astype(jnp.int32)
    iids = item_ids.astype(jnp.int32)
    uw, iw = _gather_sc(uids, iids, user_table, item_table)
    u8 = lax.bitcast_convert_type(uw, jnp.bfloat16).reshape(B, D)
    i8 = lax.bitcast_convert_type(iw, jnp.bfloat16).reshape(B, D)
    order = jnp.asarray(_PACK_ORDER)
    wu = W[:D][order].astype(jnp.bfloat16)
    wi = W[D:][order].astype(jnp.bfloat16)
    return _mm_tc(u8, i8, wu, wi, b.reshape(1, C))


# speedup vs baseline: 1.1330x; 1.1330x over previous
import functools

import numpy as np

import jax
import jax.numpy as jnp
from jax import lax
from jax.experimental import pallas as pl
from jax.experimental.pallas import tpu as pltpu
from jax.experimental.pallas import tpu_sc as plsc

B = 16384
D = 768
C = 5
NC = 2
NS = 16
NW = NC * NS
BPW = B // NW
CHUNK = 16
NCHUNK = BPW // CHUNK
NG = D // 32
CD = CHUNK * D
SW = CD // 2

_PACK_ORDER = np.arange(D).reshape(NG, 2, 16).transpose(0, 2, 1).reshape(-1)


def _gather_sc(user_ids, item_ids, user_table, item_table):
    mesh = plsc.VectorSubcoreMesh(core_axis_name="c", subcore_axis_name="s")

    @functools.partial(
        pl.kernel,
        mesh=mesh,
        out_type=[
            jax.ShapeDtypeStruct((B * D // 2,), jnp.float32),
            jax.ShapeDtypeStruct((B * D // 2,), jnp.float32),
        ],
        scratch_types=[
            pltpu.VMEM((BPW,), jnp.int32),
            pltpu.VMEM((BPW,), jnp.int32),
            pltpu.VMEM((2, CHUNK, D), jnp.float32),
            pltpu.VMEM((2, CHUNK, D), jnp.float32),
            pltpu.VMEM((2 * SW,), jnp.float32),
            pltpu.VMEM((2 * SW,), jnp.float32),
            [pltpu.SemaphoreType.DMA] * 2,
            [pltpu.SemaphoreType.DMA] * 2,
            [pltpu.SemaphoreType.DMA] * 2,
            [pltpu.SemaphoreType.DMA] * 2,
        ],
    )
    def k(uid_hbm, iid_hbm, utab_hbm, itab_hbm, uout_hbm, iout_hbm,
          uidx, iidx, ubuf, ibuf, ustage, istage, gsem_u, gsem_i,
          osem_u, osem_i):
        wid = lax.axis_index("s") * NC + lax.axis_index("c")
        base = wid * BPW
        pltpu.sync_copy(uid_hbm.at[pl.ds(base, BPW)], uidx)
        pltpu.sync_copy(iid_hbm.at[pl.ds(base, BPW)], iidx)

        def start_gather(c, bsl):
            off = pl.multiple_of(c * CHUNK, CHUNK)
            pltpu.async_copy(utab_hbm.at[uidx.at[pl.ds(off, CHUNK)]],
                             ubuf.at[bsl], gsem_u[bsl])
            pltpu.async_copy(itab_hbm.at[iidx.at[pl.ds(off, CHUNK)]],
                             ibuf.at[bsl], gsem_i[bsl])

        def wait_gather(bsl):
            pltpu.make_async_copy(utab_hbm.at[uidx.at[pl.ds(0, CHUNK)]],
                                  ubuf.at[bsl], gsem_u[bsl]).wait()
            pltpu.make_async_copy(itab_hbm.at[iidx.at[pl.ds(0, CHUNK)]],
                                  ibuf.at[bsl], gsem_i[bsl]).wait()

        def start_out(c, bsl):
            off = pl.multiple_of((base + c * CHUNK) * D // 2, SW)
            pltpu.async_copy(ustage.at[pl.ds(bsl * SW, SW)],
                             uout_hbm.at[pl.ds(off, SW)], osem_u[bsl])
            pltpu.async_copy(istage.at[pl.ds(bsl * SW, SW)],
                             iout_hbm.at[pl.ds(off, SW)], osem_i[bsl])

        def wait_out(bsl):
            pltpu.make_async_copy(ustage.at[pl.ds(bsl * SW, SW)],
                                  uout_hbm.at[pl.ds(0, SW)],
                                  osem_u[bsl]).wait()
            pltpu.make_async_copy(istage.at[pl.ds(bsl * SW, SW)],
                                  iout_hbm.at[pl.ds(0, SW)],
                                  osem_i[bsl]).wait()

        def convert_chunk(bsl):
            def row_body(r):
                so = bsl * SW + r * (D // 2)

                def cvt(buf, stage):
                    for g in range(NG):
                        lo = buf[bsl, r, pl.ds(32 * g, 16)]
                        hi = buf[bsl, r, pl.ds(32 * g + 16, 16)]
                        blo = lax.bitcast_convert_type(lo, jnp.uint32)
                        bhi = lax.bitcast_convert_type(hi, jnp.uint32)
                        word = ((bhi & jnp.uint32(0xFFFF0000))
                                | lax.shift_right_logical(
                                    blo, jnp.uint32(16)))
                        stage[pl.ds(so + 16 * g, 16)] = (
                            lax.bitcast_convert_type(word, jnp.float32))

                cvt(ubuf, ustage)
                cvt(ibuf, istage)

            plsc.parallel_loop(0, CHUNK, unroll=2)(row_body)

        start_gather(0, 0)
        start_gather(1, 1)

        def gbody(g, _):
            for par in range(2):
                c = g * 2 + par
                wait_gather(par)

                @pl.when(c >= 2)
                def _():
                    wait_out(par)

                convert_chunk(par)

                @pl.when(c + 2 < NCHUNK)
                def _():
                    start_gather(c + 2, par)

                start_out(c, par)
            return 0

        lax.fori_loop(0, NCHUNK // 2, gbody, 0)
        wait_out(0)
        wait_out(1)

    return k(user_ids, item_ids, user_table, item_table)


BM = 2048


def _mm_body(u_ref, i_ref, wu_ref, wi_ref, b_ref, o_ref):
    acc = jnp.dot(u_ref[...], wu_ref[...], preferred_element_type=jnp.float32)
    acc = acc + jnp.dot(i_ref[...], wi_ref[...],
                        preferred_element_type=jnp.float32)
    o_ref[...] = acc + b_ref[...]


def _mm_tc(u_rep, i_rep, wu, wi, b2d):
    return pl.pallas_call(
        _mm_body,
        grid=(B // BM,),
        in_specs=[
            pl.BlockSpec((BM, D), lambda m: (m, 0)),
            pl.BlockSpec((BM, D), lambda m: (m, 0)),
            pl.BlockSpec((D, C), lambda m: (0, 0)),
            pl.BlockSpec((D, C), lambda m: (0, 0)),
            pl.BlockSpec((1, C), lambda m: (0, 0)),
        ],
        out_specs=pl.BlockSpec((BM, C), lambda m: (m, 0)),
        out_shape=jax.ShapeDtypeStruct((B, C), jnp.float32),
    )(u_rep, i_rep, wu, wi, b2d)


def kernel(user_ids, item_ids, user_table, item_table, W, b):
    uids = user_ids.astype(jnp.int32)
    iids = item_ids.astype(jnp.int32)
    uw, iw = _gather_sc(uids, iids, user_table, item_table)
    u8 = lax.bitcast_convert_type(uw, jnp.bfloat16).reshape(B, D)
    i8 = lax.bitcast_convert_type(iw, jnp.bfloat16).reshape(B, D)
    order = jnp.asarray(_PACK_ORDER)
    wu = W[:D][order].astype(jnp.bfloat16)
    wi = W[D:][order].astype(jnp.bfloat16)
    return _mm_tc(u8, i8, wu, wi, b.reshape(1, C))


# speedup vs baseline: 4.7630x; 4.2038x over previous
import functools

import numpy as np

import jax
import jax.numpy as jnp
from jax import lax
from jax.experimental import pallas as pl
from jax.experimental.pallas import tpu as pltpu
from jax.experimental.pallas import tpu_sc as plsc

B = 16384
D = 768
C = 5
NC = 2
NS = 16
NW = NC * NS
BPW = B // NW
CHUNK = 16
NCHUNK = BPW // CHUNK
NG = D // 32
CD = CHUNK * D
SW = CD // 2

_W = np.arange(D // 2)
_IDX_LO = 32 * (_W // 16) + (_W % 16)
_IDX_HI = _IDX_LO + 16


def _gather_sc(user_ids, item_ids, user_table, item_table):
    mesh = plsc.VectorSubcoreMesh(core_axis_name="c", subcore_axis_name="s")

    @functools.partial(
        pl.kernel,
        mesh=mesh,
        out_type=[
            jax.ShapeDtypeStruct((B, D // 2), jnp.float32),
            jax.ShapeDtypeStruct((B, D // 2), jnp.float32),
        ],
        scratch_types=[
            pltpu.VMEM((BPW,), jnp.int32),
            pltpu.VMEM((BPW,), jnp.int32),
            pltpu.VMEM((2, CHUNK, D), jnp.float32),
            pltpu.VMEM((2, CHUNK, D), jnp.float32),
            pltpu.VMEM((2 * CHUNK, D // 2), jnp.float32),
            pltpu.VMEM((2 * CHUNK, D // 2), jnp.float32),
            [pltpu.SemaphoreType.DMA] * 2,
            [pltpu.SemaphoreType.DMA] * 2,
            [pltpu.SemaphoreType.DMA] * 2,
            [pltpu.SemaphoreType.DMA] * 2,
        ],
    )
    def k(uid_hbm, iid_hbm, utab_hbm, itab_hbm, uout_hbm, iout_hbm,
          uidx, iidx, ubuf, ibuf, ustage, istage, gsem_u, gsem_i,
          osem_u, osem_i):
        wid = lax.axis_index("s") * NC + lax.axis_index("c")
        base = wid * BPW
        pltpu.sync_copy(uid_hbm.at[pl.ds(base, BPW)], uidx)
        pltpu.sync_copy(iid_hbm.at[pl.ds(base, BPW)], iidx)

        def start_gather(c, bsl):
            off = pl.multiple_of(c * CHUNK, CHUNK)
            pltpu.async_copy(utab_hbm.at[uidx.at[pl.ds(off, CHUNK)]],
                             ubuf.at[bsl], gsem_u[bsl])
            pltpu.async_copy(itab_hbm.at[iidx.at[pl.ds(off, CHUNK)]],
                             ibuf.at[bsl], gsem_i[bsl])

        def wait_gather(bsl):
            pltpu.make_async_copy(utab_hbm.at[uidx.at[pl.ds(0, CHUNK)]],
                                  ubuf.at[bsl], gsem_u[bsl]).wait()
            pltpu.make_async_copy(itab_hbm.at[iidx.at[pl.ds(0, CHUNK)]],
                                  ibuf.at[bsl], gsem_i[bsl]).wait()

        def start_out(c, bsl):
            off = pl.multiple_of(base + c * CHUNK, CHUNK)
            pltpu.async_copy(ustage.at[pl.ds(bsl * CHUNK, CHUNK)],
                             uout_hbm.at[pl.ds(off, CHUNK)], osem_u[bsl])
            pltpu.async_copy(istage.at[pl.ds(bsl * CHUNK, CHUNK)],
                             iout_hbm.at[pl.ds(off, CHUNK)], osem_i[bsl])

        def wait_out(bsl):
            pltpu.make_async_copy(ustage.at[pl.ds(bsl * CHUNK, CHUNK)],
                                  uout_hbm.at[pl.ds(0, CHUNK)],
                                  osem_u[bsl]).wait()
            pltpu.make_async_copy(istage.at[pl.ds(bsl * CHUNK, CHUNK)],
                                  iout_hbm.at[pl.ds(0, CHUNK)],
                                  osem_i[bsl]).wait()

        def convert_chunk(bsl):
            def row_body(r):
                sr = bsl * CHUNK + r

                def cvt(buf, stage):
                    for g in range(NG):
                        lo = buf[bsl, r, pl.ds(32 * g, 16)]
                        hi = buf[bsl, r, pl.ds(32 * g + 16, 16)]
                        blo = lax.bitcast_convert_type(lo, jnp.uint32)
                        bhi = lax.bitcast_convert_type(hi, jnp.uint32)
                        word = ((bhi & jnp.uint32(0xFFFF0000))
                                | lax.shift_right_logical(
                                    blo, jnp.uint32(16)))
                        stage[sr, pl.ds(16 * g, 16)] = (
                            lax.bitcast_convert_type(word, jnp.float32))

                cvt(ubuf, ustage)
                cvt(ibuf, istage)

            plsc.parallel_loop(0, CHUNK, unroll=2)(row_body)

        start_gather(0, 0)
        start_gather(1, 1)

        def gbody(g, _):
            for par in range(2):
                c = g * 2 + par
                wait_gather(par)

                @pl.when(c >= 2)
                def _():
                    wait_out(par)

                convert_chunk(par)

                @pl.when(c + 2 < NCHUNK)
                def _():
                    start_gather(c + 2, par)

                start_out(c, par)
            return 0

        lax.fori_loop(0, NCHUNK // 2, gbody, 0)
        wait_out(0)
        wait_out(1)

    return k(user_ids, item_ids, user_table, item_table)


BM = 2048


def _unpack_words(w_ref):
    words = lax.bitcast_convert_type(w_ref[...], jnp.uint32)
    lo = lax.bitcast_convert_type(
        lax.shift_left(words, jnp.uint32(16)), jnp.float32)
    hi = lax.bitcast_convert_type(
        words & jnp.uint32(0xFFFF0000), jnp.float32)
    return lo, hi


def _mm_body(u_ref, i_ref, wul_ref, wuh_ref, wil_ref, wih_ref, b_ref, o_ref):
    ulo, uhi = _unpack_words(u_ref)
    ilo, ihi = _unpack_words(i_ref)
    acc = jnp.dot(ulo, wul_ref[...], preferred_element_type=jnp.float32)
    acc = acc + jnp.dot(uhi, wuh_ref[...], preferred_element_type=jnp.float32)
    acc = acc + jnp.dot(ilo, wil_ref[...], preferred_element_type=jnp.float32)
    acc = acc + jnp.dot(ihi, wih_ref[...], preferred_element_type=jnp.float32)
    o_ref[...] = acc + b_ref[...]


def _mm_tc(u_words, i_words, wul, wuh, wil, wih, b2d):
    hw = D // 2
    return pl.pallas_call(
        _mm_body,
        grid=(B // BM,),
        in_specs=[
            pl.BlockSpec((BM, hw), lambda m: (m, 0)),
            pl.BlockSpec((BM, hw), lambda m: (m, 0)),
            pl.BlockSpec((hw, C), lambda m: (0, 0)),
            pl.BlockSpec((hw, C), lambda m: (0, 0)),
            pl.BlockSpec((hw, C), lambda m: (0, 0)),
            pl.BlockSpec((hw, C), lambda m: (0, 0)),
            pl.BlockSpec((1, C), lambda m: (0, 0)),
        ],
        out_specs=pl.BlockSpec((BM, C), lambda m: (m, 0)),
        out_shape=jax.ShapeDtypeStruct((B, C), jnp.float32),
    )(u_words, i_words, wul, wuh, wil, wih, b2d)


def kernel(user_ids, item_ids, user_table, item_table, W, b):
    uids = user_ids.astype(jnp.int32)
    iids = item_ids.astype(jnp.int32)
    uw, iw = _gather_sc(uids, iids, user_table, item_table)
    ilo = jnp.asarray(_IDX_LO)
    ihi = jnp.asarray(_IDX_HI)
    wu, wi_ = W[:D], W[D:]
    return _mm_tc(uw, iw, wu[ilo], wu[ihi], wi_[ilo], wi_[ihi],
                  b.reshape(1, C))


# speedup vs baseline: 5.4580x; 1.1459x over previous
import functools

import numpy as np

import jax
import jax.numpy as jnp
from jax import lax
from jax.experimental import pallas as pl
from jax.experimental.pallas import tpu as pltpu
from jax.experimental.pallas import tpu_sc as plsc

B = 16384
D = 768
C = 5
NC = 2
NS = 16
NW = NC * NS
BPW = B // NW
CHUNK = 16
NCHUNK = BPW // CHUNK
NG = D // 32
CD = CHUNK * D
SW = CD // 2

_W = np.arange(D // 2)
_IDX_LO = 32 * (_W // 16) + (_W % 16)
_IDX_HI = _IDX_LO + 16


def _gather_sc(user_ids, item_ids, user_table, item_table):
    mesh = plsc.VectorSubcoreMesh(core_axis_name="c", subcore_axis_name="s")

    @functools.partial(
        pl.kernel,
        mesh=mesh,
        out_type=[
            jax.ShapeDtypeStruct((B, D // 2), jnp.float32),
            jax.ShapeDtypeStruct((B, D // 2), jnp.float32),
        ],
        scratch_types=[
            pltpu.VMEM((BPW,), jnp.int32),
            pltpu.VMEM((BPW,), jnp.int32),
            pltpu.VMEM((2, CHUNK, D), jnp.float32),
            pltpu.VMEM((2, CHUNK, D), jnp.float32),
            pltpu.VMEM((2 * CHUNK, D // 2), jnp.float32),
            pltpu.VMEM((2 * CHUNK, D // 2), jnp.float32),
            [pltpu.SemaphoreType.DMA] * 2,
            [pltpu.SemaphoreType.DMA] * 2,
            [pltpu.SemaphoreType.DMA] * 2,
            [pltpu.SemaphoreType.DMA] * 2,
        ],
    )
    def k(uid_hbm, iid_hbm, utab_hbm, itab_hbm, uout_hbm, iout_hbm,
          uidx, iidx, ubuf, ibuf, ustage, istage, gsem_u, gsem_i,
          osem_u, osem_i):
        wid = lax.axis_index("s") * NC + lax.axis_index("c")
        base = wid * BPW
        pltpu.sync_copy(uid_hbm.at[pl.ds(base, BPW)], uidx)
        pltpu.sync_copy(iid_hbm.at[pl.ds(base, BPW)], iidx)

        def start_gather(c, bsl):
            off = pl.multiple_of(c * CHUNK, CHUNK)
            pltpu.async_copy(utab_hbm.at[uidx.at[pl.ds(off, CHUNK)]],
                             ubuf.at[bsl], gsem_u[bsl])
            pltpu.async_copy(itab_hbm.at[iidx.at[pl.ds(off, CHUNK)]],
                             ibuf.at[bsl], gsem_i[bsl])

        def wait_gather(bsl):
            pltpu.make_async_copy(utab_hbm.at[uidx.at[pl.ds(0, CHUNK)]],
                                  ubuf.at[bsl], gsem_u[bsl]).wait()
            pltpu.make_async_copy(itab_hbm.at[iidx.at[pl.ds(0, CHUNK)]],
                                  ibuf.at[bsl], gsem_i[bsl]).wait()

        def start_out(c, bsl):
            off = pl.multiple_of(base + c * CHUNK, CHUNK)
            pltpu.async_copy(ustage.at[pl.ds(bsl * CHUNK, CHUNK)],
                             uout_hbm.at[pl.ds(off, CHUNK)], osem_u[bsl])
            pltpu.async_copy(istage.at[pl.ds(bsl * CHUNK, CHUNK)],
                             iout_hbm.at[pl.ds(off, CHUNK)], osem_i[bsl])

        def wait_out(bsl):
            pltpu.make_async_copy(ustage.at[pl.ds(bsl * CHUNK, CHUNK)],
                                  uout_hbm.at[pl.ds(0, CHUNK)],
                                  osem_u[bsl]).wait()
            pltpu.make_async_copy(istage.at[pl.ds(bsl * CHUNK, CHUNK)],
                                  iout_hbm.at[pl.ds(0, CHUNK)],
                                  osem_i[bsl]).wait()

        def convert_chunk(bsl):
            def row_body(r):
                sr = bsl * CHUNK + r

                def cvt(buf, stage):
                    for g in range(NG):
                        lo = buf[bsl, r, pl.ds(32 * g, 16)]
                        hi = buf[bsl, r, pl.ds(32 * g + 16, 16)]
                        blo = lax.bitcast_convert_type(lo, jnp.uint32)
                        bhi = lax.bitcast_convert_type(hi, jnp.uint32)
                        word = ((bhi & jnp.uint32(0xFFFF0000))
                                | lax.shift_right_logical(
                                    blo, jnp.uint32(16)))
                        stage[sr, pl.ds(16 * g, 16)] = (
                            lax.bitcast_convert_type(word, jnp.float32))

                cvt(ubuf, ustage)
                cvt(ibuf, istage)

            plsc.parallel_loop(0, CHUNK, unroll=4)(row_body)

        start_gather(0, 0)
        start_gather(1, 1)

        def gbody(g, _):
            for par in range(2):
                c = g * 2 + par
                wait_gather(par)

                @pl.when(c >= 2)
                def _():
                    wait_out(par)

                convert_chunk(par)

                @pl.when(c + 2 < NCHUNK)
                def _():
                    start_gather(c + 2, par)

                start_out(c, par)
            return 0

        lax.fori_loop(0, NCHUNK // 2, gbody, 0)
        wait_out(0)
        wait_out(1)

    return k(user_ids, item_ids, user_table, item_table)


BM = 2048


def _unpack_words(w_ref):
    words = lax.bitcast_convert_type(w_ref[...], jnp.uint32)
    lo = lax.bitcast_convert_type(
        lax.shift_left(words, jnp.uint32(16)), jnp.float32)
    hi = lax.bitcast_convert_type(
        words & jnp.uint32(0xFFFF0000), jnp.float32)
    return lo, hi


def _mm_body(u_ref, i_ref, wul_ref, wuh_ref, wil_ref, wih_ref, b_ref, o_ref):
    ulo, uhi = _unpack_words(u_ref)
    ilo, ihi = _unpack_words(i_ref)
    acc = jnp.dot(ulo, wul_ref[...], preferred_element_type=jnp.float32)
    acc = acc + jnp.dot(uhi, wuh_ref[...], preferred_element_type=jnp.float32)
    acc = acc + jnp.dot(ilo, wil_ref[...], preferred_element_type=jnp.float32)
    acc = acc + jnp.dot(ihi, wih_ref[...], preferred_element_type=jnp.float32)
    o_ref[...] = acc + b_ref[...]


def _mm_tc(u_words, i_words, wul, wuh, wil, wih, b2d):
    hw = D // 2
    return pl.pallas_call(
        _mm_body,
        grid=(B // BM,),
        in_specs=[
            pl.BlockSpec((BM, hw), lambda m: (m, 0)),
            pl.BlockSpec((BM, hw), lambda m: (m, 0)),
            pl.BlockSpec((hw, C), lambda m: (0, 0)),
            pl.BlockSpec((hw, C), lambda m: (0, 0)),
            pl.BlockSpec((hw, C), lambda m: (0, 0)),
            pl.BlockSpec((hw, C), lambda m: (0, 0)),
            pl.BlockSpec((1, C), lambda m: (0, 0)),
        ],
        out_specs=pl.BlockSpec((BM, C), lambda m: (m, 0)),
        out_shape=jax.ShapeDtypeStruct((B, C), jnp.float32),
    )(u_words, i_words, wul, wuh, wil, wih, b2d)


def kernel(user_ids, item_ids, user_table, item_table, W, b):
    uids = user_ids.astype(jnp.int32)
    iids = item_ids.astype(jnp.int32)
    uw, iw = _gather_sc(uids, iids, user_table, item_table)
    ilo = jnp.asarray(_IDX_LO)
    ihi = jnp.asarray(_IDX_HI)
    wu, wi_ = W[:D], W[D:]
    return _mm_tc(uw, iw, wu[ilo], wu[ihi], wi_[ilo], wi_[ihi],
                  b.reshape(1, C))
